# Initial kernel scaffold; baseline (speedup 1.0000x reference)
import jax
import jax.numpy as jnp
from jax.experimental import pallas as pl


def kernel(x, edge_index, edge_attr, batch, W1s, b1s, W2s, b2s, Wrs, brs, prelu_a, gammas, betas, Wh1, bh1, Wh2, bh2):
    raise NotImplementedError("write your pallas kernel here")


# speedup vs baseline: 2.8588x; 2.8588x over previous
import functools

import jax
import jax.numpy as jnp
from jax import lax
from jax.experimental import pallas as pl
from jax.experimental.pallas import tpu as pltpu
from jax.experimental.pallas import tpu_sc as plsc

N_NODES = 10000
N_EDGES = 320000
D_FEAT = 128
D_EDGE = 16
HID = 128
OUTDIM = 10
NUM_GRAPHS = 64

NC = 2
NS = 16
NW = NC * NS
EK = 128
NCHUNK = N_EDGES // EK
CPT = (NCHUNK + NW - 1) // NW
ROWS_MAIN = 632
ROWS_LAST = N_NODES - 15 * ROWS_MAIN

_sc_mesh = plsc.VectorSubcoreMesh(core_axis_name="c", subcore_axis_name="s")


@functools.partial(
    pl.kernel,
    mesh=_sc_mesh,
    out_type=jax.ShapeDtypeStruct((NC, N_NODES, HID), jnp.float32),
    scratch_types=[
        pltpu.VMEM((EK,), jnp.int32),
        pltpu.VMEM((EK,), jnp.int32),
        pltpu.VMEM((EK, HID), jnp.float32),
        pltpu.VMEM((EK, HID), jnp.float32),
        pltpu.VMEM_SHARED((N_NODES, HID), jnp.float32),
        pltpu.SemaphoreType.DMA,
    ],
)
def _edge_agg_sc(pre_hbm, eap_hbm, src_hbm, dst_hbm, zeros_hbm, out_hbm,
                 src_v, dst_v, rows_v, eap_v, acc_sh, sem):
    cid = lax.axis_index("c")
    sid = lax.axis_index("s")
    wid = sid * NC + cid

    row0 = pl.multiple_of(sid * ROWS_MAIN, 8)

    @pl.when(sid < NS - 1)
    def _():
        pltpu.sync_copy(zeros_hbm.at[pl.ds(row0, ROWS_MAIN)],
                        acc_sh.at[pl.ds(row0, ROWS_MAIN)])

    @pl.when(sid == NS - 1)
    def _():
        pltpu.sync_copy(zeros_hbm.at[pl.ds(row0, ROWS_LAST)],
                        acc_sh.at[pl.ds(row0, ROWS_LAST)])

    plsc.subcore_barrier()

    def chunk_body(j, carry):
        chunk = j * NW + wid

        @pl.when(chunk < NCHUNK)
        def _():
            base = pl.multiple_of(chunk * EK, EK)
            pltpu.sync_copy(src_hbm.at[pl.ds(base, EK)], src_v)
            pltpu.sync_copy(dst_hbm.at[pl.ds(base, EK)], dst_v)
            pltpu.async_copy(pre_hbm.at[src_v], rows_v, sem).wait()
            pltpu.sync_copy(eap_hbm.at[pl.ds(base, EK)], eap_v)

            def row_body(i, c2):
                for c in range(HID // 16):
                    sl = pl.ds(c * 16, 16)
                    rows_v[i, sl] = jnp.maximum(rows_v[i, sl] + eap_v[i, sl],
                                                0.0)
                return c2
            lax.fori_loop(0, EK, row_body, 0)
            pltpu.sync_copy(rows_v, acc_sh.at[dst_v], add=True)
        return carry

    lax.fori_loop(0, CPT, chunk_body, 0)
    plsc.subcore_barrier()

    @pl.when(sid < NS - 1)
    def _():
        pltpu.sync_copy(acc_sh.at[pl.ds(row0, ROWS_MAIN)],
                        out_hbm.at[cid, pl.ds(row0, ROWS_MAIN)])

    @pl.when(sid == NS - 1)
    def _():
        pltpu.sync_copy(acc_sh.at[pl.ds(row0, ROWS_LAST)],
                        out_hbm.at[cid, pl.ds(row0, ROWS_LAST)])


@functools.partial(
    pl.kernel,
    mesh=_sc_mesh,
    out_type=jax.ShapeDtypeStruct((NC, N_NODES, HID), jnp.float32),
    scratch_types=[
        pltpu.VMEM((EK,), jnp.int32),
        pltpu.VMEM((EK, HID), jnp.float32),
        pltpu.VMEM_SHARED((N_NODES, HID), jnp.float32),
        pltpu.SemaphoreType.DMA,
    ],
)
def _degree_sc(dst_hbm, zeros_hbm, out_hbm, dst_v, ones_v, acc_sh, sem):
    cid = lax.axis_index("c")
    sid = lax.axis_index("s")
    wid = sid * NC + cid

    def fill_body(i, carry):
        for c in range(HID // 16):
            ones_v[i, pl.ds(c * 16, 16)] = jnp.full((16,), 1.0, jnp.float32)
        return carry
    lax.fori_loop(0, EK, fill_body, 0)

    row0 = pl.multiple_of(sid * ROWS_MAIN, 8)

    @pl.when(sid < NS - 1)
    def _():
        pltpu.sync_copy(zeros_hbm.at[pl.ds(row0, ROWS_MAIN)],
                        acc_sh.at[pl.ds(row0, ROWS_MAIN)])

    @pl.when(sid == NS - 1)
    def _():
        pltpu.sync_copy(zeros_hbm.at[pl.ds(row0, ROWS_LAST)],
                        acc_sh.at[pl.ds(row0, ROWS_LAST)])

    plsc.subcore_barrier()

    def chunk_body(j, carry):
        chunk = j * NW + wid

        @pl.when(chunk < NCHUNK)
        def _():
            base = pl.multiple_of(chunk * EK, EK)
            pltpu.sync_copy(dst_hbm.at[pl.ds(base, EK)], dst_v)
            pltpu.sync_copy(ones_v, acc_sh.at[dst_v], add=True)
        return carry

    lax.fori_loop(0, CPT, chunk_body, 0)
    plsc.subcore_barrier()

    @pl.when(sid < NS - 1)
    def _():
        pltpu.sync_copy(acc_sh.at[pl.ds(row0, ROWS_MAIN)],
                        out_hbm.at[cid, pl.ds(row0, ROWS_MAIN)])

    @pl.when(sid == NS - 1)
    def _():
        pltpu.sync_copy(acc_sh.at[pl.ds(row0, ROWS_LAST)],
                        out_hbm.at[cid, pl.ds(row0, ROWS_LAST)])


def _pre_body(h_ref, w_ref, o_ref):
    o_ref[...] = jnp.dot(h_ref[...], w_ref[...],
                         preferred_element_type=jnp.float32)


def _pre_tc(h, w1a):
    return pl.pallas_call(
        _pre_body,
        out_shape=jax.ShapeDtypeStruct((N_NODES, HID), jnp.float32),
    )(h, w1a)


EBLK = 3200


def _eap_body(ea_ref, w_ref, b_ref, o_ref):
    o_ref[...] = (jnp.dot(ea_ref[...], w_ref[...],
                          preferred_element_type=jnp.float32)
                  + b_ref[...])


def _eap_tc(ea, w1b, b1):
    return pl.pallas_call(
        _eap_body,
        grid=(N_EDGES // EBLK,),
        in_specs=[
            pl.BlockSpec((EBLK, D_EDGE), lambda i: (i, 0)),
            pl.BlockSpec((D_EDGE, HID), lambda i: (0, 0)),
            pl.BlockSpec((1, HID), lambda i: (0, 0)),
        ],
        out_specs=pl.BlockSpec((EBLK, HID), lambda i: (i, 0)),
        out_shape=jax.ShapeDtypeStruct((N_EDGES, HID), jnp.float32),
    )(ea, w1b, b1)


def _node_body(h_ref, a0_ref, a1_ref, cnt_ref, w2_ref, b2_ref, wr_ref,
               br_ref, pa_ref, g_ref, beta_ref, o_ref):
    seg = a0_ref[...] + a1_ref[...]
    cnt = cnt_ref[...]
    aggm = jnp.dot(seg, w2_ref[...], preferred_element_type=jnp.float32)
    aggm = (aggm + cnt * b2_ref[...]) / jnp.maximum(cnt, 1.0)
    hh = (jnp.dot(h_ref[...], wr_ref[...], preferred_element_type=jnp.float32)
          + br_ref[...] + aggm)
    a = pa_ref[0, 0]
    hh = jnp.where(hh >= 0, hh, a * hh)
    mu = jnp.mean(hh, axis=0, keepdims=True)
    var = jnp.mean((hh - mu) ** 2, axis=0, keepdims=True)
    o_ref[...] = (hh - mu) * lax.rsqrt(var + 1e-5) * g_ref[...] + beta_ref[...]


def _node_tc(h, a0, a1, cnt2d, w2, b2, wr, br, pa, g, beta):
    return pl.pallas_call(
        _node_body,
        out_shape=jax.ShapeDtypeStruct((N_NODES, HID), jnp.float32),
    )(h, a0, a1, cnt2d, w2, b2, wr, br, pa, g, beta)


def _head_body(h_ref, b_ref, wh1_ref, bh1_ref, wh2_ref, bh2_ref, o_ref):
    batch = b_ref[...]
    gids = lax.broadcasted_iota(jnp.int32, (1, NUM_GRAPHS), 1)
    onehot = (batch == gids).astype(jnp.float32)
    psum = lax.dot_general(onehot, h_ref[...], (((0,), (0,)), ((), ())),
                           preferred_element_type=jnp.float32)
    ones = jnp.ones((N_NODES, 1), jnp.float32)
    pcnt = lax.dot_general(onehot, ones, (((0,), (0,)), ((), ())),
                           preferred_element_type=jnp.float32)
    pooled = psum / jnp.maximum(pcnt, 1.0)
    z = jnp.maximum(
        jnp.dot(pooled, wh1_ref[...], preferred_element_type=jnp.float32)
        + bh1_ref[...], 0.0)
    o_ref[...] = (jnp.dot(z, wh2_ref[...], preferred_element_type=jnp.float32)
                  + bh2_ref[...])


def _head_tc(h, batch2d, wh1, bh1, wh2, bh2):
    return pl.pallas_call(
        _head_body,
        out_shape=jax.ShapeDtypeStruct((NUM_GRAPHS, OUTDIM), jnp.float32),
    )(h, batch2d, wh1, bh1, wh2, bh2)


def kernel(x, edge_index, edge_attr, batch, W1s, b1s, W2s, b2s, Wrs, brs,
           prelu_a, gammas, betas, Wh1, bh1, Wh2, bh2):
    src = edge_index[0].astype(jnp.int32)
    dst = edge_index[1].astype(jnp.int32)
    zeros_nh = jnp.zeros((N_NODES, HID), jnp.float32)

    deg = _degree_sc(dst, zeros_nh)
    cnt2d = (deg[0, :, 0] + deg[1, :, 0]).reshape(N_NODES, 1)

    h = x
    for i in range(4):
        w1a = W1s[i, :D_FEAT]
        w1b = W1s[i, D_FEAT:]
        pre = _pre_tc(h, w1a)
        eap = _eap_tc(edge_attr, w1b, b1s[i].reshape(1, HID))
        parts = _edge_agg_sc(pre, eap, src, dst, zeros_nh)
        h = _node_tc(h, parts[0], parts[1], cnt2d, W2s[i],
                     b2s[i].reshape(1, HID), Wrs[i], brs[i].reshape(1, HID),
                     prelu_a[i].reshape(1, 1), gammas[i].reshape(1, HID),
                     betas[i].reshape(1, HID))

    return _head_tc(h, batch.astype(jnp.int32).reshape(N_NODES, 1),
                    Wh1, bh1.reshape(1, HID), Wh2, bh2.reshape(1, OUTDIM))
